# baseline (speedup 1.0000x reference)
import jax
import jax.numpy as jnp
from jax.experimental import pallas as pl
from jax.experimental.pallas import tpu as pltpu

_A = 3
_C = 85
_NET = 608.0
_HP = 2


def _decode_kernel(anchors_ref, x_ref, o_ref):
    hp = pl.program_id(0)
    x = x_ref[...]
    HP, W, B, CC = x_ref.shape

    lshape = (1, 1, 1, CC)
    cg = jax.lax.broadcasted_iota(jnp.int32, lshape, 3)
    cl = cg % _C
    ai = cg // _C
    m23 = (cl == 2) | (cl == 3)
    is2 = cl == 2
    awh = jnp.where(
        ai == 0,
        jnp.where(is2, anchors_ref[0, 0], anchors_ref[0, 1]),
        jnp.where(
            ai == 1,
            jnp.where(is2, anchors_ref[1, 0], anchors_ref[1, 1]),
            jnp.where(is2, anchors_ref[2, 0], anchors_ref[2, 1]),
        ),
    ) * (1.0 / _NET)
    sc = jnp.where(cl < 2, 1.0 / W, 1.0)
    m0 = (cl == 0).astype(jnp.float32)
    m1 = (cl == 1).astype(jnp.float32)

    gshape = (HP, W, 1, CC)
    gi = jax.lax.broadcasted_iota(jnp.int32, gshape, 1).astype(jnp.float32)
    gj = (hp * HP + jax.lax.broadcasted_iota(jnp.int32, gshape, 0)).astype(
        jnp.float32
    )
    g = gi * m0 + gj * m1

    s = jax.nn.sigmoid(x)
    e = jnp.exp(x)
    y = jnp.where(m23, e * awh, (s + g) * sc)
    for a in range(_A):
        ya = y[:, :, :, a * _C:(a + 1) * _C]
        za = jnp.transpose(ya, (2, 0, 1, 3))
        o_ref[:, a, :, :] = za.reshape(B, _HP * W, _C)


def kernel(output, anchors):
    B, CC, H, W = output.shape
    assert CC == _A * _C
    xt = jnp.transpose(output, (2, 3, 0, 1))
    nsplit = 2
    bs = B // nsplit
    halves = []
    for bg in range(nsplit):
        o = pl.pallas_call(
            _decode_kernel,
            grid=(H // _HP,),
            in_specs=[
                pl.BlockSpec(memory_space=pltpu.SMEM),
                pl.BlockSpec(
                    (_HP, W, bs, CC), lambda hp, bg=bg: (hp, 0, bg, 0)
                ),
            ],
            out_specs=pl.BlockSpec(
                (bs, _A, _HP * W, _C), lambda hp: (0, 0, hp, 0)
            ),
            out_shape=jax.ShapeDtypeStruct((bs, _A, H * W, _C), jnp.float32),
            compiler_params=pltpu.CompilerParams(
                dimension_semantics=("arbitrary",)
            ),
        )(anchors, xt)
        halves.append(o.reshape(bs, _A * H * W, _C))
    return jnp.concatenate(halves, axis=0)


# speedup vs baseline: 1.3363x; 1.3363x over previous
import jax
import jax.numpy as jnp
from jax.experimental import pallas as pl
from jax.experimental.pallas import tpu as pltpu

_A = 3
_C = 85
_NET = 608.0
_HP = 2


def _decode_kernel(anchors_ref, x_ref, o_ref):
    hp = pl.program_id(0)
    x = x_ref[...]
    HP, W, B, CC = x_ref.shape

    lshape = (1, 1, 1, CC)
    cg = jax.lax.broadcasted_iota(jnp.int32, lshape, 3)
    cl = cg % _C
    ai = cg // _C
    m23 = (cl == 2) | (cl == 3)
    is2 = cl == 2
    awh = jnp.where(
        ai == 0,
        jnp.where(is2, anchors_ref[0, 0], anchors_ref[0, 1]),
        jnp.where(
            ai == 1,
            jnp.where(is2, anchors_ref[1, 0], anchors_ref[1, 1]),
            jnp.where(is2, anchors_ref[2, 0], anchors_ref[2, 1]),
        ),
    ) * (1.0 / _NET)
    sc = jnp.where(cl < 2, 1.0 / W, 1.0)
    m0 = (cl == 0).astype(jnp.float32)
    m1 = (cl == 1).astype(jnp.float32)

    gshape = (HP, W, 1, CC)
    gi = jax.lax.broadcasted_iota(jnp.int32, gshape, 1).astype(jnp.float32)
    gj = (hp * HP + jax.lax.broadcasted_iota(jnp.int32, gshape, 0)).astype(
        jnp.float32
    )
    g = gi * m0 + gj * m1

    s = jnp.tanh(x * 0.5) * 0.5 + 0.5
    e = jnp.exp(x)
    y = jnp.where(m23, e * awh, (s + g) * sc)
    for a in range(_A):
        ya = y[:, :, :, a * _C:(a + 1) * _C]
        za = jnp.transpose(ya, (2, 0, 1, 3))
        o_ref[:, a, :, :] = za.reshape(B, _HP * W, _C)


def kernel(output, anchors):
    B, CC, H, W = output.shape
    assert CC == _A * _C
    xt = jnp.transpose(output, (2, 3, 0, 1))
    out = pl.pallas_call(
        _decode_kernel,
        grid=(H // _HP,),
        in_specs=[
            pl.BlockSpec(memory_space=pltpu.SMEM),
            pl.BlockSpec((_HP, W, B, CC), lambda hp: (hp, 0, 0, 0)),
        ],
        out_specs=pl.BlockSpec(
            (B, _A, _HP * W, _C), lambda hp: (0, 0, hp, 0)
        ),
        out_shape=jax.ShapeDtypeStruct((B, _A, H * W, _C), jnp.float32),
        compiler_params=pltpu.CompilerParams(
            dimension_semantics=("arbitrary",)
        ),
    )(anchors, xt)
    return out.reshape(B, _A * H * W, _C)


# speedup vs baseline: 1.3725x; 1.0270x over previous
import jax
import jax.numpy as jnp
from jax.experimental import pallas as pl
from jax.experimental.pallas import tpu as pltpu

_A = 3
_C = 85
_NET = 608.0
_HP = 4


def _decode_kernel(anchors_ref, x_ref, o_ref):
    hp = pl.program_id(0)
    x = x_ref[...]
    HP, W, B, CC = x_ref.shape

    lshape = (1, 1, 1, CC)
    cg = jax.lax.broadcasted_iota(jnp.int32, lshape, 3)
    cl = cg % _C
    ai = cg // _C
    m23 = (cl == 2) | (cl == 3)
    is2 = cl == 2
    awh = jnp.where(
        ai == 0,
        jnp.where(is2, anchors_ref[0, 0], anchors_ref[0, 1]),
        jnp.where(
            ai == 1,
            jnp.where(is2, anchors_ref[1, 0], anchors_ref[1, 1]),
            jnp.where(is2, anchors_ref[2, 0], anchors_ref[2, 1]),
        ),
    ) * (1.0 / _NET)
    sc = jnp.where(cl < 2, 1.0 / W, 1.0)
    m0 = (cl == 0).astype(jnp.float32)
    m1 = (cl == 1).astype(jnp.float32)

    gshape = (HP, W, 1, CC)
    gi = jax.lax.broadcasted_iota(jnp.int32, gshape, 1).astype(jnp.float32)
    gj = (hp * HP + jax.lax.broadcasted_iota(jnp.int32, gshape, 0)).astype(
        jnp.float32
    )
    g = gi * m0 + gj * m1

    s = jnp.tanh(x * 0.5) * 0.5 + 0.5
    e = jnp.exp(x)
    y = jnp.where(m23, e * awh, (s + g) * sc)
    for a in range(_A):
        ya = y[:, :, :, a * _C:(a + 1) * _C]
        za = jnp.transpose(ya, (2, 0, 1, 3))
        o_ref[:, a, :, :] = za.reshape(B, _HP * W, _C)


def kernel(output, anchors):
    B, CC, H, W = output.shape
    assert CC == _A * _C
    xt = jnp.transpose(output, (2, 3, 0, 1))
    out = pl.pallas_call(
        _decode_kernel,
        grid=(H // _HP,),
        in_specs=[
            pl.BlockSpec(memory_space=pltpu.SMEM),
            pl.BlockSpec((_HP, W, B, CC), lambda hp: (hp, 0, 0, 0)),
        ],
        out_specs=pl.BlockSpec(
            (B, _A, _HP * W, _C), lambda hp: (0, 0, hp, 0)
        ),
        out_shape=jax.ShapeDtypeStruct((B, _A, H * W, _C), jnp.float32),
        compiler_params=pltpu.CompilerParams(
            dimension_semantics=("arbitrary",)
        ),
    )(anchors, xt)
    return out.reshape(B, _A * H * W, _C)
